# Initial kernel scaffold; baseline (speedup 1.0000x reference)
import functools

import jax
import jax.numpy as jnp
from jax import lax
from jax.experimental import pallas as pl
from jax.experimental.pallas import tpu as pltpu

SEQ, N, E = 24, 10000, 640000
DYN, STAT, HID, EMB = 8, 16, 64, 32
BN = 2000

_INTERPRET = False


def _dinv_of(deg):
    return jnp.where(deg > 0, lax.rsqrt(deg), 0.0)


def _enc_body(dyn_ref, stat_ref, deg_ref,
              dw1, db1, dw2, db2, sw1, sb1, sw2, sb2,
              wih, whh, lb,
              t1s_ref, raw_ref, cenc_ref, state_ref):
    bn = stat_ref.shape[0]
    d = dyn_ref[...].reshape(SEQ * bn, DYN)
    e1 = jnp.tanh(d @ dw1[...] + db1[...])
    dyn_e = (e1 @ dw2[...] + db2[...]).reshape(SEQ, bn, EMB)
    s1 = jnp.tanh(stat_ref[...] @ sw1[...] + sb1[...])
    stat_e = s1 @ sw2[...] + sb2[...]

    wih_v = wih[...]
    whh_v = whh[...]
    lb_v = lb[...]

    def step(t, hc):
        h, c = hc
        x_t = jnp.concatenate([dyn_e[t], stat_e], axis=-1)
        g = x_t @ wih_v + h @ whh_v + lb_v
        i_, f_, g_, o_ = jnp.split(g, 4, axis=-1)
        c = jax.nn.sigmoid(f_) * c + jax.nn.sigmoid(i_) * jnp.tanh(g_)
        h = jax.nn.sigmoid(o_) * jnp.tanh(c)
        return (h, c)

    h0 = jnp.zeros((bn, HID), jnp.float32)
    h, c = lax.fori_loop(0, SEQ, step, (h0, h0))

    x_dec = jnp.concatenate([dyn_e[SEQ - 1], stat_e], axis=-1)
    raw = jnp.concatenate([x_dec, h], axis=-1)
    deg = deg_ref[0, :] + deg_ref[1, :]
    dinv = _dinv_of(deg)[:, None]
    t1s_ref[...] = dinv * raw
    raw_ref[...] = raw
    cenc_ref[...] = c
    state_ref[...] = stat_e


def _encoder(dyn, stat, deg_p, P):
    grid = N // BN
    full = lambda s: pl.BlockSpec(s, lambda i: (0,) * len(s))
    out = pl.pallas_call(
        _enc_body,
        grid=(grid,),
        in_specs=[
            pl.BlockSpec((SEQ, BN, DYN), lambda i: (0, i, 0)),
            pl.BlockSpec((BN, STAT), lambda i: (i, 0)),
            pl.BlockSpec((2, BN), lambda i: (0, i)),
            full((DYN, 64)), full((1, 64)), full((64, EMB)), full((1, EMB)),
            full((STAT, 64)), full((1, 64)), full((64, EMB)), full((1, EMB)),
            full((2 * EMB, 4 * HID)), full((HID, 4 * HID)), full((1, 4 * HID)),
        ],
        out_specs=[
            pl.BlockSpec((BN, 2 * HID), lambda i: (i, 0)),
            pl.BlockSpec((BN, 2 * HID), lambda i: (i, 0)),
            pl.BlockSpec((BN, HID), lambda i: (i, 0)),
            pl.BlockSpec((BN, EMB), lambda i: (i, 0)),
        ],
        out_shape=[
            jax.ShapeDtypeStruct((N, 2 * HID), jnp.float32),
            jax.ShapeDtypeStruct((N, 2 * HID), jnp.float32),
            jax.ShapeDtypeStruct((N, HID), jnp.float32),
            jax.ShapeDtypeStruct((N, EMB), jnp.float32),
        ],
        interpret=_INTERPRET,
    )(dyn, stat, deg_p,
      P['dyn_w1'], P['dyn_b1'].reshape(1, -1), P['dyn_w2'], P['dyn_b2'].reshape(1, -1),
      P['stat_w1'], P['stat_b1'].reshape(1, -1), P['stat_w2'], P['stat_b2'].reshape(1, -1),
      P['lstm_wih'].T, P['lstm_whh'].T,
      (P['lstm_bih'] + P['lstm_bhh']).reshape(1, -1))
    return out


def _cell1_body(raw_ref, cenc_ref, txp_ref, deg_ref,
                w0x, w1x, w0h, w1h, b_ref,
                h1_ref, h1s_ref):
    raw = raw_ref[...]
    x_dec = raw[:, :2 * EMB]
    h_enc = raw[:, 2 * EMB:]
    deg = deg_ref[0, :] + deg_ref[1, :]
    dinv = _dinv_of(deg)[:, None]
    tx = (-dinv) * (txp_ref[0] + txp_ref[1])
    tx_x = tx[:, :2 * EMB]
    tx_h = tx[:, 2 * EMB:]
    gates = (x_dec @ w0x[...] + tx_x @ w1x[...]
             + h_enc @ w0h[...] + tx_h @ w1h[...] + b_ref[...])
    i_, f_, g_, o_ = jnp.split(gates, 4, axis=-1)
    c1 = jax.nn.sigmoid(f_) * cenc_ref[...] + jax.nn.sigmoid(i_) * jnp.tanh(g_)
    h1 = jax.nn.sigmoid(o_) * jnp.tanh(c1)
    h1_ref[...] = h1
    h1s_ref[...] = dinv * h1


def _cell1(raw, c_enc, txp, deg_p, w0x, w1x, w0h, w1h, b):
    grid = N // BN
    full = lambda s: pl.BlockSpec(s, lambda i: (0,) * len(s))
    return pl.pallas_call(
        _cell1_body,
        grid=(grid,),
        in_specs=[
            pl.BlockSpec((BN, 2 * HID), lambda i: (i, 0)),
            pl.BlockSpec((BN, HID), lambda i: (i, 0)),
            pl.BlockSpec((2, BN, 2 * HID), lambda i: (0, i, 0)),
            pl.BlockSpec((2, BN), lambda i: (0, i)),
            full((HID, 4 * HID)), full((HID, 4 * HID)),
            full((HID, 4 * HID)), full((HID, 4 * HID)), full((1, 4 * HID)),
        ],
        out_specs=[
            pl.BlockSpec((BN, HID), lambda i: (i, 0)),
            pl.BlockSpec((BN, HID), lambda i: (i, 0)),
        ],
        out_shape=[
            jax.ShapeDtypeStruct((N, HID), jnp.float32),
            jax.ShapeDtypeStruct((N, HID), jnp.float32),
        ],
        interpret=_INTERPRET,
    )(raw, c_enc, txp, deg_p, w0x, w1x, w0h, w1h, b)


def _cell2_body(h1_ref, txp_ref, deg_ref, stat_ref,
                w0x, w1x, b_ref, dw1, db1, dw2, db2, sw1, sb1, sw2, sb2,
                dis_ref, swi_ref):
    deg = deg_ref[0, :] + deg_ref[1, :]
    dinv = _dinv_of(deg)[:, None]
    tx_h1 = (-dinv) * (txp_ref[0] + txp_ref[1])
    gates = h1_ref[...] @ w0x[...] + tx_h1 @ w1x[...] + b_ref[...]
    i_, f_, g_, o_ = jnp.split(gates, 4, axis=-1)
    c2 = jax.nn.sigmoid(i_) * jnp.tanh(g_)
    h2 = jax.nn.sigmoid(o_) * jnp.tanh(c2)
    ds = jnp.concatenate([h2, stat_ref[...]], axis=-1)
    dis = jax.nn.relu(ds @ dw1[...] + db1[...]) @ dw2[...] + db2[...]
    swi = jax.nn.sigmoid(jnp.tanh(ds @ sw1[...] + sb1[...]) @ sw2[...] + sb2[...])
    dis_ref[...] = dis[:, 0]
    swi_ref[...] = swi[:, 0]


def _cell2(h1, txp2, deg_p, stat_e, w0x, w1x, b, P):
    grid = N // BN
    full = lambda s: pl.BlockSpec(s, lambda i: (0,) * len(s))
    return pl.pallas_call(
        _cell2_body,
        grid=(grid,),
        in_specs=[
            pl.BlockSpec((BN, HID), lambda i: (i, 0)),
            pl.BlockSpec((2, BN, HID), lambda i: (0, i, 0)),
            pl.BlockSpec((2, BN), lambda i: (0, i)),
            pl.BlockSpec((BN, EMB), lambda i: (i, 0)),
            full((HID, 4 * HID)), full((HID, 4 * HID)), full((1, 4 * HID)),
            full((HID + EMB, HID // 2)), full((1, HID // 2)),
            full((HID // 2, 1)), full((1, 1)),
            full((HID + EMB, 64)), full((1, 64)), full((64, 1)), full((1, 1)),
        ],
        out_specs=[
            pl.BlockSpec((BN,), lambda i: (i,)),
            pl.BlockSpec((BN,), lambda i: (i,)),
        ],
        out_shape=[
            jax.ShapeDtypeStruct((N,), jnp.float32),
            jax.ShapeDtypeStruct((N,), jnp.float32),
        ],
        interpret=_INTERPRET,
    )(h1, txp2, deg_p, stat_e, w0x, w1x, b,
      P['dis_w1'], P['dis_b1'].reshape(1, -1), P['dis_w2'], P['dis_b2'].reshape(1, -1),
      P['swi_w1'], P['swi_b1'].reshape(1, -1), P['swi_w2'], P['swi_b2'].reshape(1, -1))


def _deg_pass(src, ew):
    deg = jnp.zeros((N,), jnp.float32).at[src].add(ew)
    return jnp.stack([deg, jnp.zeros_like(deg)])


def _scatter_pass(table, src, dst, ew):
    acc = jnp.zeros((N, table.shape[1]), jnp.float32)
    acc = acc.at[dst].add(ew[:, None] * table[src])
    return jnp.stack([acc, jnp.zeros_like(acc)])


def kernel(dynamic_seq, static_feat, edge_index, edge_weight, params):
    P = params
    src, dst = edge_index[0], edge_index[1]

    deg_p = _deg_pass(src, edge_weight)

    t1s, raw, c_enc, stat_e = _encoder(dynamic_seq, static_feat, deg_p, P)

    txp = _scatter_pass(t1s, src, dst, edge_weight)

    def cat_w(layer, idx):
        return jnp.concatenate([P[layer + g + '_w' + idx] for g in 'ifgo'], axis=1)

    b1 = jnp.concatenate([P['d1x' + g + '_b'] + P['d1h' + g + '_b'] for g in 'ifgo'])
    b2 = jnp.concatenate([P['d2x' + g + '_b'] + P['d2h' + g + '_b'] for g in 'ifgo'])

    h1, h1s = _cell1(raw, c_enc, txp, deg_p,
                     cat_w('d1x', '0'), cat_w('d1x', '1'),
                     cat_w('d1h', '0'), cat_w('d1h', '1'), b1.reshape(1, -1))

    txp2 = _scatter_pass(h1s, src, dst, edge_weight)

    dis, swi = _cell2(h1, txp2, deg_p, stat_e,
                      cat_w('d2x', '0'), cat_w('d2x', '1'), b2.reshape(1, -1), P)
    return dis, swi


# speedup vs baseline: 3.2128x; 3.2128x over previous
import functools

import jax
import jax.numpy as jnp
from jax import lax
from jax.experimental import pallas as pl
from jax.experimental.pallas import tpu as pltpu

SEQ, N, E = 24, 10000, 640000
DYN, STAT, HID, EMB = 8, 16, 64, 32
BN = 2000

_INTERPRET = False


def _dinv_of(deg):
    return jnp.where(deg > 0, lax.rsqrt(deg), 0.0)


def _enc_body(dyn_ref, stat_ref, deg_ref,
              dw1, db1, dw2, db2, sw1, sb1, sw2, sb2,
              wih, whh, lb,
              t1s_ref, raw_ref, cenc_ref, state_ref):
    bn = stat_ref.shape[0]
    dyn = dyn_ref[...]
    s1 = jnp.tanh(stat_ref[...] @ sw1[...] + sb1[...])
    stat_e = s1 @ sw2[...] + sb2[...]

    dw1_v, db1_v, dw2_v, db2_v = dw1[...], db1[...], dw2[...], db2[...]
    wih_v = wih[...]
    whh_v = whh[...]
    lb_v = lb[...]

    def dyn_emb(t):
        e1 = jnp.tanh(dyn[:, t * DYN:(t + 1) * DYN] @ dw1_v + db1_v)
        return e1 @ dw2_v + db2_v

    h = jnp.zeros((bn, HID), jnp.float32)
    c = jnp.zeros((bn, HID), jnp.float32)
    x_t = None
    for t in range(SEQ):
        x_t = jnp.concatenate([dyn_emb(t), stat_e], axis=-1)
        g = x_t @ wih_v + h @ whh_v + lb_v
        i_, f_, g_, o_ = jnp.split(g, 4, axis=-1)
        c = jax.nn.sigmoid(f_) * c + jax.nn.sigmoid(i_) * jnp.tanh(g_)
        h = jax.nn.sigmoid(o_) * jnp.tanh(c)

    x_dec = x_t
    raw = jnp.concatenate([x_dec, h], axis=-1)
    deg = deg_ref[:, 0] + deg_ref[:, 1]
    dinv = _dinv_of(deg)[:, None]
    t1s_ref[...] = dinv * raw
    raw_ref[...] = raw
    cenc_ref[...] = c
    state_ref[...] = stat_e


def _encoder(dyn2, stat, deg_p, P):
    grid = N // BN
    full = lambda s: pl.BlockSpec(s, lambda i: (0,) * len(s))
    out = pl.pallas_call(
        _enc_body,
        grid=(grid,),
        in_specs=[
            pl.BlockSpec((BN, SEQ * DYN), lambda i: (i, 0)),
            pl.BlockSpec((BN, STAT), lambda i: (i, 0)),
            pl.BlockSpec((BN, 2), lambda i: (i, 0)),
            full((DYN, 64)), full((1, 64)), full((64, EMB)), full((1, EMB)),
            full((STAT, 64)), full((1, 64)), full((64, EMB)), full((1, EMB)),
            full((2 * EMB, 4 * HID)), full((HID, 4 * HID)), full((1, 4 * HID)),
        ],
        out_specs=[
            pl.BlockSpec((BN, 2 * HID), lambda i: (i, 0)),
            pl.BlockSpec((BN, 2 * HID), lambda i: (i, 0)),
            pl.BlockSpec((BN, HID), lambda i: (i, 0)),
            pl.BlockSpec((BN, EMB), lambda i: (i, 0)),
        ],
        out_shape=[
            jax.ShapeDtypeStruct((N, 2 * HID), jnp.float32),
            jax.ShapeDtypeStruct((N, 2 * HID), jnp.float32),
            jax.ShapeDtypeStruct((N, HID), jnp.float32),
            jax.ShapeDtypeStruct((N, EMB), jnp.float32),
        ],
        interpret=_INTERPRET,
    )(dyn2, stat, deg_p,
      P['dyn_w1'], P['dyn_b1'].reshape(1, -1), P['dyn_w2'], P['dyn_b2'].reshape(1, -1),
      P['stat_w1'], P['stat_b1'].reshape(1, -1), P['stat_w2'], P['stat_b2'].reshape(1, -1),
      P['lstm_wih'].T, P['lstm_whh'].T,
      (P['lstm_bih'] + P['lstm_bhh']).reshape(1, -1))
    return out


def _cell1_body(raw_ref, cenc_ref, txp_ref, deg_ref,
                w0x, w1x, w0h, w1h, b_ref,
                h1_ref, h1s_ref):
    raw = raw_ref[...]
    x_dec = raw[:, :2 * EMB]
    h_enc = raw[:, 2 * EMB:]
    deg = deg_ref[:, 0] + deg_ref[:, 1]
    dinv = _dinv_of(deg)[:, None]
    tx = (-dinv) * (txp_ref[0] + txp_ref[1])
    tx_x = tx[:, :2 * EMB]
    tx_h = tx[:, 2 * EMB:]
    gates = (x_dec @ w0x[...] + tx_x @ w1x[...]
             + h_enc @ w0h[...] + tx_h @ w1h[...] + b_ref[...])
    i_, f_, g_, o_ = jnp.split(gates, 4, axis=-1)
    c1 = jax.nn.sigmoid(f_) * cenc_ref[...] + jax.nn.sigmoid(i_) * jnp.tanh(g_)
    h1 = jax.nn.sigmoid(o_) * jnp.tanh(c1)
    h1_ref[...] = h1
    h1s_ref[...] = dinv * h1


def _cell1(raw, c_enc, txp, deg_p, w0x, w1x, w0h, w1h, b):
    grid = N // BN
    full = lambda s: pl.BlockSpec(s, lambda i: (0,) * len(s))
    return pl.pallas_call(
        _cell1_body,
        grid=(grid,),
        in_specs=[
            pl.BlockSpec((BN, 2 * HID), lambda i: (i, 0)),
            pl.BlockSpec((BN, HID), lambda i: (i, 0)),
            pl.BlockSpec((2, BN, 2 * HID), lambda i: (0, i, 0)),
            pl.BlockSpec((BN, 2), lambda i: (i, 0)),
            full((HID, 4 * HID)), full((HID, 4 * HID)),
            full((HID, 4 * HID)), full((HID, 4 * HID)), full((1, 4 * HID)),
        ],
        out_specs=[
            pl.BlockSpec((BN, HID), lambda i: (i, 0)),
            pl.BlockSpec((BN, HID), lambda i: (i, 0)),
        ],
        out_shape=[
            jax.ShapeDtypeStruct((N, HID), jnp.float32),
            jax.ShapeDtypeStruct((N, HID), jnp.float32),
        ],
        interpret=_INTERPRET,
    )(raw, c_enc, txp, deg_p, w0x, w1x, w0h, w1h, b)


def _cell2_body(h1_ref, txp_ref, deg_ref, stat_ref,
                w0x, w1x, b_ref, dw1, db1, dw2, db2, sw1, sb1, sw2, sb2,
                dis_ref, swi_ref):
    deg = deg_ref[:, 0] + deg_ref[:, 1]
    dinv = _dinv_of(deg)[:, None]
    tx_h1 = (-dinv) * (txp_ref[0] + txp_ref[1])
    gates = h1_ref[...] @ w0x[...] + tx_h1 @ w1x[...] + b_ref[...]
    i_, f_, g_, o_ = jnp.split(gates, 4, axis=-1)
    c2 = jax.nn.sigmoid(i_) * jnp.tanh(g_)
    h2 = jax.nn.sigmoid(o_) * jnp.tanh(c2)
    ds = jnp.concatenate([h2, stat_ref[...]], axis=-1)
    dis = jax.nn.relu(ds @ dw1[...] + db1[...]) @ dw2[...] + db2[...]
    swi = jax.nn.sigmoid(jnp.tanh(ds @ sw1[...] + sb1[...]) @ sw2[...] + sb2[...])
    dis_ref[...] = dis
    swi_ref[...] = swi


def _cell2(h1, txp2, deg_p, stat_e, w0x, w1x, b, P):
    grid = N // BN
    full = lambda s: pl.BlockSpec(s, lambda i: (0,) * len(s))
    return pl.pallas_call(
        _cell2_body,
        grid=(grid,),
        in_specs=[
            pl.BlockSpec((BN, HID), lambda i: (i, 0)),
            pl.BlockSpec((2, BN, HID), lambda i: (0, i, 0)),
            pl.BlockSpec((BN, 2), lambda i: (i, 0)),
            pl.BlockSpec((BN, EMB), lambda i: (i, 0)),
            full((HID, 4 * HID)), full((HID, 4 * HID)), full((1, 4 * HID)),
            full((HID + EMB, HID // 2)), full((1, HID // 2)),
            full((HID // 2, 1)), full((1, 1)),
            full((HID + EMB, 64)), full((1, 64)), full((64, 1)), full((1, 1)),
        ],
        out_specs=[
            pl.BlockSpec((BN, 1), lambda i: (i, 0)),
            pl.BlockSpec((BN, 1), lambda i: (i, 0)),
        ],
        out_shape=[
            jax.ShapeDtypeStruct((N, 1), jnp.float32),
            jax.ShapeDtypeStruct((N, 1), jnp.float32),
        ],
        interpret=_INTERPRET,
    )(h1, txp2, deg_p, stat_e, w0x, w1x, b,
      P['dis_w1'], P['dis_b1'].reshape(1, -1), P['dis_w2'], P['dis_b2'].reshape(1, -1),
      P['swi_w1'], P['swi_b1'].reshape(1, -1), P['swi_w2'], P['swi_b2'].reshape(1, -1))


def _deg_pass(src, ew):
    deg = jnp.zeros((N,), jnp.float32).at[src].add(ew)
    return jnp.stack([deg, jnp.zeros_like(deg)], axis=1)


def _scatter_pass(table, src, dst, ew):
    acc = jnp.zeros((N, table.shape[1]), jnp.float32)
    acc = acc.at[dst].add(ew[:, None] * table[src])
    return jnp.stack([acc, jnp.zeros_like(acc)])


def kernel(dynamic_seq, static_feat, edge_index, edge_weight, params):
    P = params
    src, dst = edge_index[0], edge_index[1]

    deg_p = _deg_pass(src, edge_weight)

    dyn2 = dynamic_seq.transpose(1, 0, 2).reshape(N, SEQ * DYN)
    t1s, raw, c_enc, stat_e = _encoder(dyn2, static_feat, deg_p, P)

    txp = _scatter_pass(t1s, src, dst, edge_weight)

    def cat_w(layer, idx):
        return jnp.concatenate([P[layer + g + '_w' + idx] for g in 'ifgo'], axis=1)

    b1 = jnp.concatenate([P['d1x' + g + '_b'] + P['d1h' + g + '_b'] for g in 'ifgo'])
    b2 = jnp.concatenate([P['d2x' + g + '_b'] + P['d2h' + g + '_b'] for g in 'ifgo'])

    h1, h1s = _cell1(raw, c_enc, txp, deg_p,
                     cat_w('d1x', '0'), cat_w('d1x', '1'),
                     cat_w('d1h', '0'), cat_w('d1h', '1'), b1.reshape(1, -1))

    txp2 = _scatter_pass(h1s, src, dst, edge_weight)

    dis, swi = _cell2(h1, txp2, deg_p, stat_e,
                      cat_w('d2x', '0'), cat_w('d2x', '1'), b2.reshape(1, -1), P)
    return dis[:, 0], swi[:, 0]


# speedup vs baseline: 13.9833x; 4.3524x over previous
import functools

import jax
import jax.numpy as jnp
from jax import lax
from jax.experimental import pallas as pl
from jax.experimental.pallas import tpu as pltpu
from jax.experimental.pallas import tpu_sc as plsc

SEQ, N, E = 24, 10000, 640000
DYN, STAT, HID, EMB = 8, 16, 64, 32
BN = 2000

NC, NS = 2, 16
NW = NC * NS
CH = 128
NCHUNK = 158
EPT = CH * NCHUNK
EP = EPT * NW
_MAGIC = 0x5F3759DF

_INTERPRET = False
_SYNC_RING = True


def _enc_body(dyn_ref, stat_ref,
              dw1, db1, dw2, db2, sw1, sb1, sw2, sb2,
              wih, whh, lb,
              xdec_ref, henc_ref, cenc_ref, state_ref):
    bn = stat_ref.shape[0]
    dyn = dyn_ref[...]
    s1 = jnp.tanh(stat_ref[...] @ sw1[...] + sb1[...])
    stat_e = s1 @ sw2[...] + sb2[...]

    dw1_v, db1_v, dw2_v, db2_v = dw1[...], db1[...], dw2[...], db2[...]
    wih_v = wih[...]
    whh_v = whh[...]
    lb_v = lb[...]

    def dyn_emb(t):
        e1 = jnp.tanh(dyn[:, t * DYN:(t + 1) * DYN] @ dw1_v + db1_v)
        return e1 @ dw2_v + db2_v

    h = jnp.zeros((bn, HID), jnp.float32)
    c = jnp.zeros((bn, HID), jnp.float32)
    x_t = None
    for t in range(SEQ):
        x_t = jnp.concatenate([dyn_emb(t), stat_e], axis=-1)
        g = x_t @ wih_v + h @ whh_v + lb_v
        i_, f_, g_, o_ = jnp.split(g, 4, axis=-1)
        c = jax.nn.sigmoid(f_) * c + jax.nn.sigmoid(i_) * jnp.tanh(g_)
        h = jax.nn.sigmoid(o_) * jnp.tanh(c)

    xdec_ref[...] = x_t
    henc_ref[...] = h
    cenc_ref[...] = c
    state_ref[...] = stat_e


def _encoder(dyn2, stat, P):
    grid = N // BN
    full = lambda s: pl.BlockSpec(s, lambda i: (0,) * len(s))
    out = pl.pallas_call(
        _enc_body,
        grid=(grid,),
        in_specs=[
            pl.BlockSpec((BN, SEQ * DYN), lambda i: (i, 0)),
            pl.BlockSpec((BN, STAT), lambda i: (i, 0)),
            full((DYN, 64)), full((1, 64)), full((64, EMB)), full((1, EMB)),
            full((STAT, 64)), full((1, 64)), full((64, EMB)), full((1, EMB)),
            full((2 * EMB, 4 * HID)), full((HID, 4 * HID)), full((1, 4 * HID)),
        ],
        out_specs=[
            pl.BlockSpec((BN, 2 * EMB), lambda i: (i, 0)),
            pl.BlockSpec((BN, HID), lambda i: (i, 0)),
            pl.BlockSpec((BN, HID), lambda i: (i, 0)),
            pl.BlockSpec((BN, EMB), lambda i: (i, 0)),
        ],
        out_shape=[
            jax.ShapeDtypeStruct((N, 2 * EMB), jnp.float32),
            jax.ShapeDtypeStruct((N, HID), jnp.float32),
            jax.ShapeDtypeStruct((N, HID), jnp.float32),
            jax.ShapeDtypeStruct((N, EMB), jnp.float32),
        ],
        interpret=_INTERPRET,
    )(dyn2, stat,
      P['dyn_w1'], P['dyn_b1'].reshape(1, -1), P['dyn_w2'], P['dyn_b2'].reshape(1, -1),
      P['stat_w1'], P['stat_b1'].reshape(1, -1), P['stat_w2'], P['stat_b2'].reshape(1, -1),
      P['lstm_wih'].T, P['lstm_whh'].T,
      (P['lstm_bih'] + P['lstm_bhh']).reshape(1, -1))
    return out


def _bcast16(vec, j):
    return jnp.take(vec, jnp.full((16,), j, dtype=jnp.int32))


def _zero_vregs(ref, nrows, ncols):
    def zrow(r, carry):
        for k in range(ncols // 16):
            ref[r, pl.ds(k * 16, 16)] = jnp.zeros((16,), jnp.float32)
        return carry
    lax.fori_loop(0, nrows, zrow, 0)


def _tile_rows(s):
    return s * 624


def _fill_slice(dst, zrows, base, sizes):
    off = 0
    for sz in sizes:
        pltpu.sync_copy(zrows.at[pl.ds(0, sz)], dst.at[pl.ds(base + off, sz)])
        off += sz


def _scale_rows(rows_v, ew_v, ch):
    def grp(g, carry):
        ewv = ew_v[ch, pl.ds(g * 16, 16)]
        for j in range(16):
            bc = _bcast16(ewv, j)
            e = g * 16 + j
            for k in range(HID // 16):
                sl = pl.ds(k * 16, 16)
                rows_v[e, sl] = rows_v[e, sl] * bc
        return carry
    lax.fori_loop(0, CH // 16, grp, 0)


def _scatter_ring(table_hbm, src_v, dst_v, ew_v, rowsA, rowsB, accum,
                  semgA, semgB, semsA, semsB):
    def g_start(rows, sem, ch):
        pltpu.async_copy(table_hbm.at[src_v.at[ch]], rows, sem)

    def g_wait(rows, sem):
        pltpu.make_async_copy(table_hbm.at[src_v.at[0]], rows, sem).wait()

    def s_start(rows, sem, ch):
        pltpu.async_copy(rows, accum.at[dst_v.at[ch]], sem, add=True)

    def s_wait(rows, sem):
        pltpu.make_async_copy(rows, accum.at[dst_v.at[0]], sem).wait()

    if _SYNC_RING:
        def body(ch, carry):
            g_start(rowsA, semgA, ch)
            g_wait(rowsA, semgA)
            _scale_rows(rowsA, ew_v, ch)
            s_start(rowsA, semsA, ch)
            s_wait(rowsA, semsA)
            return carry

        lax.fori_loop(0, NCHUNK, body, 0)
    else:
        g_start(rowsA, semgA, 0)
        g_start(rowsB, semgB, 1)

        def body(gi, carry):
            chA = 2 * gi
            g_wait(rowsA, semgA)
            _scale_rows(rowsA, ew_v, chA)
            s_start(rowsA, semsA, chA)
            g_wait(rowsB, semgB)
            _scale_rows(rowsB, ew_v, chA + 1)
            s_start(rowsB, semsB, chA + 1)

            @pl.when(gi + 1 < NCHUNK // 2)
            def _():
                s_wait(rowsA, semsA)
                g_start(rowsA, semgA, chA + 2)
                s_wait(rowsB, semsB)
                g_start(rowsB, semgB, chA + 3)
            return carry

        lax.fori_loop(0, NCHUNK // 2, body, 0)
        s_wait(rowsA, semsA)
        s_wait(rowsB, semsB)


def _drain(accum, out_slice_fn, s):
    base = _tile_rows(s)

    @pl.when(s < NS - 1)
    def _():
        pltpu.sync_copy(accum.at[pl.ds(base, 624)], out_slice_fn(base, 624))

    @pl.when(s == NS - 1)
    def _():
        pltpu.sync_copy(accum.at[pl.ds(base, 640)], out_slice_fn(base, 640))


def _zero_accum(accum, zrows, s):
    base = _tile_rows(s)

    @pl.when(s < NS - 1)
    def _():
        _fill_slice(accum, zrows, base, (128, 128, 128, 128, 112))

    @pl.when(s == NS - 1)
    def _():
        _fill_slice(accum, zrows, base, (128, 128, 128, 128, 128))


def _sc_mesh():
    return plsc.VectorSubcoreMesh(core_axis_name="c", subcore_axis_name="s",
                                  num_cores=NC, num_subcores=NS)


def _sc_params():
    return pltpu.CompilerParams(use_tc_tiling_on_sc=False, needs_layout_passes=False, internal_scratch_in_bytes=1 << 18)


def _sc_pass1(xdec, henc, srcm, dstm, ewm):
    @functools.partial(
        pl.kernel,
        out_type=[
            jax.ShapeDtypeStruct((NC, N, HID), jnp.float32),
            jax.ShapeDtypeStruct((NC, N, HID), jnp.float32),
            jax.ShapeDtypeStruct((N,), jnp.float32),
        ],
        mesh=_sc_mesh(),
        compiler_params=_sc_params(),
        scratch_types=[
            pltpu.VMEM((NCHUNK, CH), jnp.int32),
            pltpu.VMEM((NCHUNK, CH), jnp.int32),
            pltpu.VMEM((NCHUNK, CH), jnp.float32),
            pltpu.VMEM((CH, HID), jnp.float32),
            pltpu.VMEM((CH, HID), jnp.float32),
            pltpu.VMEM((CH, HID), jnp.float32),
            pltpu.VMEM((N,), jnp.float32),
            pltpu.VMEM((640,), jnp.float32),
            pltpu.VMEM_SHARED((N,), jnp.float32),
            pltpu.VMEM_SHARED((N, HID), jnp.float32),
            pltpu.SemaphoreType.DMA,
            pltpu.SemaphoreType.DMA,
            pltpu.SemaphoreType.DMA,
            pltpu.SemaphoreType.DMA,
            pltpu.SemaphoreType.DMA,
        ],
    )
    def k(xdec_hbm, henc_hbm, src_hbm, dst_hbm, ew_hbm,
          txpA_hbm, txpB_hbm, dinv_hbm,
          src_v, dst_v, ew_v, rowsA, rowsB, zrows, dinv_v, zdeg,
          deg_sh, accum, semgA, semgB, semsA, semsB, semd):
        c = lax.axis_index("c")
        s = lax.axis_index("s")
        wid = c * NS + s
        other_wid = (1 - c) * NS + s
        base = _tile_rows(s)

        _zero_vregs(zrows, CH, HID)
        for k in range(640 // 16):
            zdeg[pl.ds(k * 16, 16)] = jnp.zeros((16,), jnp.float32)
        _zero_accum(accum, zrows, s)

        @pl.when(s < NS - 1)
        def _():
            pltpu.sync_copy(zdeg.at[pl.ds(0, 624)], deg_sh.at[pl.ds(base, 624)])

        @pl.when(s == NS - 1)
        def _():
            pltpu.sync_copy(zdeg.at[pl.ds(0, 640)], deg_sh.at[pl.ds(base, 640)])

        plsc.subcore_barrier()

        for w2 in (other_wid, wid):
            pltpu.sync_copy(src_hbm.at[w2], src_v)
            pltpu.sync_copy(ew_hbm.at[w2], ew_v)

            def dbody(ch, carry):
                pltpu.async_copy(ew_v.at[ch], deg_sh.at[src_v.at[ch]],
                                 semd, add=True).wait()
                return carry
            lax.fori_loop(0, NCHUNK, dbody, 0)
        pltpu.sync_copy(dst_hbm.at[wid], dst_v)
        plsc.subcore_barrier()

        pltpu.sync_copy(deg_sh, dinv_v)

        def rbody(i, carry):
            sl = pl.ds(i * 16, 16)
            d = dinv_v[sl]
            bits = plsc.bitcast(d, jnp.int32)
            y = plsc.bitcast(jnp.full((16,), _MAGIC, jnp.int32)
                             - lax.shift_right_arithmetic(bits, 1), jnp.float32)
            for _ in range(3):
                y = y * (1.5 - 0.5 * d * y * y)
            dinv_v[sl] = jnp.where(d > 0, y, jnp.zeros((16,), jnp.float32))
            return carry
        lax.fori_loop(0, N // 16, rbody, 0)

        def nrow(r, carry):
            for g in range(CH // 16):
                sl = pl.ds(g * 16, 16)
                sv = src_v[r, sl]
                dv = dst_v[r, sl]
                w = ew_v[r, sl]
                ew_v[r, sl] = -(plsc.load_gather(dinv_v, [sv]) * w
                                * plsc.load_gather(dinv_v, [dv]))
            return carry
        lax.fori_loop(0, NCHUNK, nrow, 0)

        @pl.when(c == 0)
        def _():
            @pl.when(s < NS - 1)
            def _():
                pltpu.sync_copy(dinv_v.at[pl.ds(base, 624)],
                                dinv_hbm.at[pl.ds(base, 624)])

            @pl.when(s == NS - 1)
            def _():
                pltpu.sync_copy(dinv_v.at[pl.ds(base, 640)],
                                dinv_hbm.at[pl.ds(base, 640)])

        _scatter_ring(xdec_hbm, src_v, dst_v, ew_v, rowsA, rowsB, accum,
                      semgA, semgB, semsA, semsB)
        plsc.subcore_barrier()
        _drain(accum, lambda b, sz: txpA_hbm.at[c, pl.ds(b, sz)], s)
        _zero_accum(accum, zrows, s)
        plsc.subcore_barrier()

        _scatter_ring(henc_hbm, src_v, dst_v, ew_v, rowsA, rowsB, accum,
                      semgA, semgB, semsA, semsB)
        plsc.subcore_barrier()
        _drain(accum, lambda b, sz: txpB_hbm.at[c, pl.ds(b, sz)], s)

    return k(xdec, henc, srcm, dstm, ewm)


def _sc_pass2(h1, srcm, dstm, ewm, dinv):
    @functools.partial(
        pl.kernel,
        out_type=jax.ShapeDtypeStruct((NC, N, HID), jnp.float32),
        mesh=_sc_mesh(),
        compiler_params=_sc_params(),
        scratch_types=[
            pltpu.VMEM((NCHUNK, CH), jnp.int32),
            pltpu.VMEM((NCHUNK, CH), jnp.int32),
            pltpu.VMEM((NCHUNK, CH), jnp.float32),
            pltpu.VMEM((CH, HID), jnp.float32),
            pltpu.VMEM((CH, HID), jnp.float32),
            pltpu.VMEM((CH, HID), jnp.float32),
            pltpu.VMEM((N,), jnp.float32),
            pltpu.VMEM_SHARED((N, HID), jnp.float32),
            pltpu.SemaphoreType.DMA,
            pltpu.SemaphoreType.DMA,
            pltpu.SemaphoreType.DMA,
            pltpu.SemaphoreType.DMA,
        ],
    )
    def k(h1_hbm, src_hbm, dst_hbm, ew_hbm, dinv_hbm, out_hbm,
          src_v, dst_v, ew_v, rowsA, rowsB, zrows, dinv_v,
          accum, semgA, semgB, semsA, semsB):
        c = lax.axis_index("c")
        s = lax.axis_index("s")
        wid = c * NS + s

        pltpu.sync_copy(src_hbm.at[wid], src_v)
        pltpu.sync_copy(dst_hbm.at[wid], dst_v)
        pltpu.sync_copy(ew_hbm.at[wid], ew_v)
        pltpu.sync_copy(dinv_hbm, dinv_v)

        def nrow(r, carry):
            for g in range(CH // 16):
                sl = pl.ds(g * 16, 16)
                sv = src_v[r, sl]
                dv = dst_v[r, sl]
                w = ew_v[r, sl]
                ew_v[r, sl] = -(plsc.load_gather(dinv_v, [sv]) * w
                                * plsc.load_gather(dinv_v, [dv]))
            return carry
        lax.fori_loop(0, NCHUNK, nrow, 0)

        _zero_vregs(zrows, CH, HID)
        _zero_accum(accum, zrows, s)
        plsc.subcore_barrier()

        _scatter_ring(h1_hbm, src_v, dst_v, ew_v, rowsA, rowsB, accum,
                      semgA, semgB, semsA, semsB)
        plsc.subcore_barrier()
        _drain(accum, lambda b, sz: out_hbm.at[c, pl.ds(b, sz)], s)

    return k(h1, srcm, dstm, ewm, dinv)


def _cell1_body(xdec_ref, henc_ref, cenc_ref, txpA_ref, txpB_ref,
                w0x, w1x, w0h, w1h, b_ref, h1_ref):
    tx_x = txpA_ref[0] + txpA_ref[1]
    tx_h = txpB_ref[0] + txpB_ref[1]
    gates = (xdec_ref[...] @ w0x[...] + tx_x @ w1x[...]
             + henc_ref[...] @ w0h[...] + tx_h @ w1h[...] + b_ref[...])
    i_, f_, g_, o_ = jnp.split(gates, 4, axis=-1)
    c1 = jax.nn.sigmoid(f_) * cenc_ref[...] + jax.nn.sigmoid(i_) * jnp.tanh(g_)
    h1_ref[...] = jax.nn.sigmoid(o_) * jnp.tanh(c1)


def _cell1(xdec, henc, c_enc, txpA, txpB, w0x, w1x, w0h, w1h, b):
    grid = N // BN
    full = lambda s: pl.BlockSpec(s, lambda i: (0,) * len(s))
    return pl.pallas_call(
        _cell1_body,
        grid=(grid,),
        in_specs=[
            pl.BlockSpec((BN, 2 * EMB), lambda i: (i, 0)),
            pl.BlockSpec((BN, HID), lambda i: (i, 0)),
            pl.BlockSpec((BN, HID), lambda i: (i, 0)),
            pl.BlockSpec((2, BN, HID), lambda i: (0, i, 0)),
            pl.BlockSpec((2, BN, HID), lambda i: (0, i, 0)),
            full((HID, 4 * HID)), full((HID, 4 * HID)),
            full((HID, 4 * HID)), full((HID, 4 * HID)), full((1, 4 * HID)),
        ],
        out_specs=pl.BlockSpec((BN, HID), lambda i: (i, 0)),
        out_shape=jax.ShapeDtypeStruct((N, HID), jnp.float32),
        interpret=_INTERPRET,
    )(xdec, henc, c_enc, txpA, txpB, w0x, w1x, w0h, w1h, b)


def _cell2_body(h1_ref, txp_ref, stat_ref,
                w0x, w1x, b_ref, dw1, db1, dw2, db2, sw1, sb1, sw2, sb2,
                dis_ref, swi_ref):
    tx_h1 = txp_ref[0] + txp_ref[1]
    gates = h1_ref[...] @ w0x[...] + tx_h1 @ w1x[...] + b_ref[...]
    i_, f_, g_, o_ = jnp.split(gates, 4, axis=-1)
    c2 = jax.nn.sigmoid(i_) * jnp.tanh(g_)
    h2 = jax.nn.sigmoid(o_) * jnp.tanh(c2)
    ds = jnp.concatenate([h2, stat_ref[...]], axis=-1)
    dis = jax.nn.relu(ds @ dw1[...] + db1[...]) @ dw2[...] + db2[...]
    swi = jax.nn.sigmoid(jnp.tanh(ds @ sw1[...] + sb1[...]) @ sw2[...] + sb2[...])
    dis_ref[...] = dis
    swi_ref[...] = swi


def _cell2(h1, txp2, stat_e, w0x, w1x, b, P):
    grid = N // BN
    full = lambda s: pl.BlockSpec(s, lambda i: (0,) * len(s))
    return pl.pallas_call(
        _cell2_body,
        grid=(grid,),
        in_specs=[
            pl.BlockSpec((BN, HID), lambda i: (i, 0)),
            pl.BlockSpec((2, BN, HID), lambda i: (0, i, 0)),
            pl.BlockSpec((BN, EMB), lambda i: (i, 0)),
            full((HID, 4 * HID)), full((HID, 4 * HID)), full((1, 4 * HID)),
            full((HID + EMB, HID // 2)), full((1, HID // 2)),
            full((HID // 2, 1)), full((1, 1)),
            full((HID + EMB, 64)), full((1, 64)), full((64, 1)), full((1, 1)),
        ],
        out_specs=[
            pl.BlockSpec((BN, 1), lambda i: (i, 0)),
            pl.BlockSpec((BN, 1), lambda i: (i, 0)),
        ],
        out_shape=[
            jax.ShapeDtypeStruct((N, 1), jnp.float32),
            jax.ShapeDtypeStruct((N, 1), jnp.float32),
        ],
        interpret=_INTERPRET,
    )(h1, txp2, stat_e, w0x, w1x, b,
      P['dis_w1'], P['dis_b1'].reshape(1, -1), P['dis_w2'], P['dis_b2'].reshape(1, -1),
      P['swi_w1'], P['swi_b1'].reshape(1, -1), P['swi_w2'], P['swi_b2'].reshape(1, -1))


def kernel(dynamic_seq, static_feat, edge_index, edge_weight, params):
    P = params
    src, dst = edge_index[0], edge_index[1]

    pad = EP - E
    pad_idx = (jnp.arange(pad, dtype=jnp.int32) % N)
    srcm = jnp.concatenate([src, pad_idx]).reshape(NW, NCHUNK, CH)
    dstm = jnp.concatenate([dst, pad_idx]).reshape(NW, NCHUNK, CH)
    ewm = jnp.concatenate([edge_weight,
                           jnp.zeros((pad,), jnp.float32)]).reshape(NW, NCHUNK, CH)

    dyn2 = dynamic_seq.transpose(1, 0, 2).reshape(N, SEQ * DYN)
    xdec, henc, c_enc, stat_e = _encoder(dyn2, static_feat, P)

    txpA, txpB, dinv = _sc_pass1(xdec, henc, srcm, dstm, ewm)

    def cat_w(layer, idx):
        return jnp.concatenate([P[layer + g + '_w' + idx] for g in 'ifgo'], axis=1)

    b1 = jnp.concatenate([P['d1x' + g + '_b'] + P['d1h' + g + '_b'] for g in 'ifgo'])
    b2 = jnp.concatenate([P['d2x' + g + '_b'] + P['d2h' + g + '_b'] for g in 'ifgo'])

    h1 = _cell1(xdec, henc, c_enc, txpA, txpB,
                cat_w('d1x', '0'), cat_w('d1x', '1'),
                cat_w('d1h', '0'), cat_w('d1h', '1'), b1.reshape(1, -1))

    txp2 = _sc_pass2(h1, srcm, dstm, ewm, dinv)

    dis, swi = _cell2(h1, txp2, stat_e,
                      cat_w('d2x', '0'), cat_w('d2x', '1'), b2.reshape(1, -1), P)
    return dis[:, 0], swi[:, 0]
